# Initial kernel scaffold; baseline (speedup 1.0000x reference)
import jax
import jax.numpy as jnp
from jax.experimental import pallas as pl


def kernel(batch_token, self_idx_batch, parent_idx_batch, root_mask, leaf_mask, start_token, end_token, V_W1, V_b1, V_W2, V_b2, E_W1, E_b1, E_W2, E_b2, p_W1, p_b1, p_W2, p_b2, c_W1, c_b1, c_W2, c_b2, aggr_W1, aggr_b1, aggr_W2, aggr_b2):
    raise NotImplementedError("write your pallas kernel here")


# speedup vs baseline: 2.3654x; 2.3654x over previous
import functools

import jax
import jax.numpy as jnp
from jax import lax
from jax.experimental import pallas as pl
from jax.experimental.pallas import tpu as pltpu
from jax.experimental.pallas import tpu_sc as plsc

N = 10000
E = 320000
D = 128
H1 = 256
H2 = 128

NC = 2
NS = 16
NW = NC * NS

EPW = E // NW
EPT = E // NS
N_PAD = 10240
NPT = N_PAD // NS
GC = 80
SCC = 80
CW = 128

_mesh = plsc.VectorSubcoreMesh(core_axis_name="c", subcore_axis_name="s")
_f32 = jnp.float32


@functools.partial(
    pl.kernel,
    out_type=(jax.ShapeDtypeStruct((E, D), _f32),
              jax.ShapeDtypeStruct((E, D), _f32)),
    mesh=_mesh,
    scratch_types=[
        pltpu.VMEM((GC,), jnp.int32),
        pltpu.VMEM((GC,), jnp.int32),
        pltpu.VMEM((GC, D), _f32),
        pltpu.VMEM((GC, D), _f32),
        pltpu.SemaphoreType.DMA,
        pltpu.SemaphoreType.DMA,
    ],
)
def _sc_gather(hidden, sidx, pidx, outs, outp,
               sidx_v, pidx_v, rows_s, rows_p, sem_s, sem_p):
    wid = lax.axis_index("s") * NC + lax.axis_index("c")
    base = wid * EPW

    def body(i, carry):
        off = base + i * GC
        pltpu.sync_copy(sidx.at[pl.ds(off, GC)], sidx_v)
        pltpu.sync_copy(pidx.at[pl.ds(off, GC)], pidx_v)
        cs = pltpu.async_copy(hidden.at[sidx_v], rows_s, sem_s)
        cp = pltpu.async_copy(hidden.at[pidx_v], rows_p, sem_p)
        cs.wait()
        cp.wait()
        pltpu.sync_copy(rows_s, outs.at[pl.ds(off, GC)])
        pltpu.sync_copy(rows_p, outp.at[pl.ds(off, GC)])
        return carry

    lax.fori_loop(0, EPW // GC, body, 0)


@functools.partial(
    pl.kernel,
    out_type=(jax.ShapeDtypeStruct((N_PAD, D), _f32),
              jax.ShapeDtypeStruct((N_PAD, D), _f32)),
    mesh=_mesh,
    scratch_types=[
        pltpu.VMEM_SHARED((N_PAD, D), _f32),
        pltpu.VMEM((SCC,), jnp.int32),
        pltpu.VMEM((SCC, D), _f32),
    ],
)
def _sc_scatter(vp, vc, sidx, pidx, zeros_nd, outp, outc, acc, idx_v, val_v):
    cid = lax.axis_index("c")
    sid = lax.axis_index("s")
    row0 = sid * NPT
    pltpu.sync_copy(zeros_nd.at[pl.ds(row0, NPT)], acc.at[pl.ds(row0, NPT)])
    plsc.subcore_barrier()

    def run(vals_hbm, idx_hbm, out_hbm):
        base = sid * EPT

        def body(i, carry):
            off = base + i * SCC
            pltpu.sync_copy(idx_hbm.at[pl.ds(off, SCC)], idx_v)
            pltpu.sync_copy(vals_hbm.at[pl.ds(off, SCC)], val_v)
            pltpu.sync_copy(val_v, acc.at[idx_v], add=True)
            return carry

        lax.fori_loop(0, EPT // SCC, body, 0)
        plsc.subcore_barrier()
        pltpu.sync_copy(acc.at[pl.ds(row0, NPT)], out_hbm.at[pl.ds(row0, NPT)])

    @pl.when(cid == 0)
    def _():
        run(vp, sidx, outp)

    @pl.when(cid == 1)
    def _():
        run(vc, pidx, outc)


@functools.partial(
    pl.kernel,
    out_type=(jax.ShapeDtypeStruct((N_PAD, CW), _f32),
              jax.ShapeDtypeStruct((N_PAD, CW), _f32)),
    mesh=_mesh,
    scratch_types=[
        pltpu.VMEM_SHARED((N_PAD, CW), _f32),
        pltpu.VMEM((SCC,), jnp.int32),
        pltpu.VMEM((SCC, CW), _f32),
    ],
)
def _sc_counts(sidx, pidx, ones_cw, zeros_cw, outs, outp, acc, idx_v, ones_v):
    cid = lax.axis_index("c")
    sid = lax.axis_index("s")
    row0 = sid * NPT
    pltpu.sync_copy(ones_cw, ones_v)
    pltpu.sync_copy(zeros_cw.at[pl.ds(row0, NPT)], acc.at[pl.ds(row0, NPT)])
    plsc.subcore_barrier()

    def run(idx_hbm, out_hbm):
        base = sid * EPT

        def body(i, carry):
            off = base + i * SCC
            pltpu.sync_copy(idx_hbm.at[pl.ds(off, SCC)], idx_v)
            pltpu.sync_copy(ones_v, acc.at[idx_v], add=True)
            return carry

        lax.fori_loop(0, EPT // SCC, body, 0)
        plsc.subcore_barrier()
        pltpu.sync_copy(acc.at[pl.ds(row0, NPT)], out_hbm.at[pl.ds(row0, NPT)])

    @pl.when(cid == 0)
    def _():
        run(sidx, outs)

    @pl.when(cid == 1)
    def _():
        run(pidx, outp)


def _relu(x):
    return jnp.maximum(x, 0.0)


def _dot(a, b):
    return jnp.dot(a, b, preferred_element_type=_f32)


def _vmlp_body(x, w1, b1, w2, b2, o):
    h = _relu(_dot(x[...], w1[...]) + b1[...])
    o[...] = _relu(_dot(h, w2[...]) + b2[...])


def _vmlp(x, w1, b1, w2, b2):
    BN = 1000
    return pl.pallas_call(
        _vmlp_body,
        grid=(N // BN,),
        in_specs=[
            pl.BlockSpec((BN, D), lambda i: (i, 0)),
            pl.BlockSpec((D, H1), lambda i: (0, 0)),
            pl.BlockSpec((1, H1), lambda i: (0, 0)),
            pl.BlockSpec((H1, H2), lambda i: (0, 0)),
            pl.BlockSpec((1, H2), lambda i: (0, 0)),
        ],
        out_specs=pl.BlockSpec((BN, H2), lambda i: (i, 0)),
        out_shape=jax.ShapeDtypeStruct((N, H2), _f32),
    )(x, w1, b1, w2, b2)


def _edge_body(s, p, w1s, w1p, b1, wp2, bp2, wc2, bc2, op, oc):
    sv = s[...]
    pv = p[...]
    h = _relu(_dot(sv, w1s[...]) + _dot(pv, w1p[...]) + b1[...])
    op[...] = _relu(_dot(h[:, :H1], wp2[...]) + bp2[...])
    oc[...] = _relu(_dot(h[:, H1:], wc2[...]) + bc2[...])


def _edge_mlp(gs, gp, w1s, w1p, b1, wp2, bp2, wc2, bc2):
    BE = 1000
    full = lambda *shape: pl.BlockSpec(shape, lambda i: (0,) * len(shape))
    return pl.pallas_call(
        _edge_body,
        grid=(E // BE,),
        in_specs=[
            pl.BlockSpec((BE, D), lambda i: (i, 0)),
            pl.BlockSpec((BE, D), lambda i: (i, 0)),
            full(D, 2 * H1),
            full(D, 2 * H1),
            full(1, 2 * H1),
            full(H1, H2),
            full(1, H2),
            full(H1, H2),
            full(1, H2),
        ],
        out_specs=[
            pl.BlockSpec((BE, H2), lambda i: (i, 0)),
            pl.BlockSpec((BE, H2), lambda i: (i, 0)),
        ],
        out_shape=[
            jax.ShapeDtypeStruct((E, H2), _f32),
            jax.ShapeDtypeStruct((E, H2), _f32),
        ],
    )(gs, gp, w1s, w1p, b1, wp2, bp2, wc2, bc2)


def _aggr_body(h, sp, sc_, cs, cp, rm, lm, st, et, wa, wb, wc, b1, w2, b2, o):
    hv = h[...]
    rcs = 1.0 / jnp.maximum(cs[...][:, 0:1], 1.0)
    rcp = 1.0 / jnp.maximum(cp[...][:, 0:1], 1.0)
    spv = sp[...] * rcs + rm[...] * st[...]
    scv = sc_[...] * rcp + lm[...] * et[...]
    h1 = _relu(_dot(hv, wa[...]) + _dot(spv, wb[...]) + _dot(scv, wc[...])
               + b1[...])
    o[...] = hv + _relu(_dot(h1, w2[...]) + b2[...])


def _aggr(h, sp, sc_, cs, cp, rm, lm, st, et, wa, wb, wc, b1, w2, b2):
    BN = 1000
    full = lambda *shape: pl.BlockSpec(shape, lambda i: (0,) * len(shape))
    return pl.pallas_call(
        _aggr_body,
        grid=(N // BN,),
        in_specs=[
            pl.BlockSpec((BN, H2), lambda i: (i, 0)),
            pl.BlockSpec((BN, H2), lambda i: (i, 0)),
            pl.BlockSpec((BN, H2), lambda i: (i, 0)),
            pl.BlockSpec((BN, CW), lambda i: (i, 0)),
            pl.BlockSpec((BN, CW), lambda i: (i, 0)),
            pl.BlockSpec((BN, 1), lambda i: (i, 0)),
            pl.BlockSpec((BN, 1), lambda i: (i, 0)),
            full(1, D),
            full(1, D),
            full(H2, H1),
            full(H2, H1),
            full(H2, H1),
            full(1, H1),
            full(H1, H2),
            full(1, H2),
        ],
        out_specs=pl.BlockSpec((BN, H2), lambda i: (i, 0)),
        out_shape=jax.ShapeDtypeStruct((N, H2), _f32),
    )(h, sp, sc_, cs, cp, rm, lm, st, et, wa, wb, wc, b1, w2, b2)


def kernel(batch_token, self_idx_batch, parent_idx_batch, root_mask, leaf_mask,
           start_token, end_token,
           V_W1, V_b1, V_W2, V_b2,
           E_W1, E_b1, E_W2, E_b2,
           p_W1, p_b1, p_W2, p_b2,
           c_W1, c_b1, c_W2, c_b2,
           aggr_W1, aggr_b1, aggr_W2, aggr_b2):
    one = jnp.ones((1, 1), _f32)
    zero = jnp.zeros((1, 1), _f32)
    edge_in = _relu(_relu(one @ E_W1 + E_b1) @ E_W2 + E_b2)
    edge_out = _relu(_relu(zero @ E_W1 + E_b1) @ E_W2 + E_b2)
    w1s = jnp.concatenate([p_W1[D:2 * D], c_W1[0:D]], axis=1)
    w1p = jnp.concatenate([p_W1[0:D], c_W1[D:2 * D]], axis=1)
    b1f = jnp.concatenate([p_b1[None, :] + edge_out @ p_W1[2 * D:],
                           c_b1[None, :] + edge_in @ c_W1[2 * D:]], axis=1)
    wa = aggr_W1[0:D]
    wb = aggr_W1[D:2 * D]
    wc = aggr_W1[2 * D:]

    rm2 = root_mask[:, None]
    lm2 = leaf_mask[:, None]
    st2 = start_token[None, :]
    et2 = end_token[None, :]
    zeros_nd = jnp.zeros((N_PAD, D), _f32)
    ones_cw = jnp.ones((SCC, CW), _f32)
    zeros_cw = jnp.zeros((N_PAD, CW), _f32)

    hidden = _vmlp(batch_token, V_W1, V_b1[None, :], V_W2, V_b2[None, :])
    cnt_s, cnt_p = _sc_counts(self_idx_batch, parent_idx_batch,
                              ones_cw, zeros_cw)
    cnt_s = cnt_s[:N]
    cnt_p = cnt_p[:N]

    for _hop in range(2):
        gs, gp = _sc_gather(hidden, self_idx_batch, parent_idx_batch)
        vp, vc = _edge_mlp(gs, gp, w1s, w1p, b1f,
                           p_W2, p_b2[None, :], c_W2, c_b2[None, :])
        sum_p, sum_c = _sc_scatter(vp, vc, self_idx_batch, parent_idx_batch,
                                   zeros_nd)
        sum_p = sum_p[:N]
        sum_c = sum_c[:N]
        hidden = _aggr(hidden, sum_p, sum_c, cnt_s, cnt_p, rm2, lm2, st2, et2,
                       wa, wb, wc, aggr_b1[None, :], aggr_W2,
                       aggr_b2[None, :])
    return hidden
